# speedup vs baseline: 3.1022x; 3.1022x over previous
import functools

import numpy as np
import jax
import jax.numpy as jnp
from jax import lax
from jax.experimental import pallas as pl
from jax.experimental.pallas import tpu as pltpu
from jax.experimental.pallas import tpu_sc as plsc

_NC, _NS, _LANES = 2, 16, 16
_NW = _NC * _NS


@functools.lru_cache(maxsize=None)
def _sinusoidal_pe(seq_len: int, d_model: int) -> np.ndarray:
    pos = np.arange(seq_len, dtype=np.float32)[:, None]
    two_i = np.arange(0, d_model, 2, dtype=np.float32)
    div = np.power(10000.0, (two_i / np.float32(d_model)).astype(np.float32))
    pe = np.zeros((seq_len, d_model), dtype=np.float32)
    pe[:, 0::2] = np.sin(pos / div)
    pe[:, 1::2] = np.cos(pos / div)
    return pe


@functools.lru_cache(maxsize=None)
def _make_embed(B: int, S: int, V: int, D: int):
    assert S % _NW == 0 and D % _LANES == 0
    s_per_w = S // _NW
    ch = min(32, s_per_w)
    assert s_per_w % ch == 0 and ch % 8 == 0
    nch = s_per_w // ch

    mesh = plsc.VectorSubcoreMesh(
        core_axis_name="c", subcore_axis_name="s",
        num_cores=_NC, num_subcores=_NS)

    def body(x_hbm, table_hbm, pe_hbm, out_hbm, idx_v, pe_v, rows_v, sem):
        cid = lax.axis_index("c")
        sid = lax.axis_index("s")
        wid = sid * _NC + cid
        s_base = wid * s_per_w
        for c in range(nch):
            spos = s_base + c * ch
            pltpu.sync_copy(pe_hbm.at[pl.ds(spos, ch)], pe_v)
            for b in range(B):
                row0 = b * S + spos
                pltpu.sync_copy(x_hbm.at[pl.ds(row0, ch)], idx_v)
                pltpu.async_copy(table_hbm.at[idx_v], rows_v, sem).wait()

                def add_row(r, _):
                    for j in range(D // _LANES):
                        sl = pl.ds(j * _LANES, _LANES)
                        plsc.addupdate(rows_v.at[r, sl], pe_v[r, sl])
                    return 0

                lax.fori_loop(0, ch, add_row, 0)
                pltpu.sync_copy(rows_v, out_hbm.at[pl.ds(row0, ch)])

    return pl.kernel(
        body,
        out_type=jax.ShapeDtypeStruct((B * S, D), jnp.float32),
        mesh=mesh,
        scratch_types=[
            pltpu.VMEM((ch,), jnp.int32),
            pltpu.VMEM((ch, D), jnp.float32),
            pltpu.VMEM((ch, D), jnp.float32),
            pltpu.SemaphoreType.DMA,
        ],
    )


def kernel(x, token_table):
    B, S = x.shape
    V, D = token_table.shape
    pe = jnp.asarray(_sinusoidal_pe(S, D))
    x_flat = x.reshape(B * S).astype(jnp.int32)
    out = _make_embed(B, S, V, D)(x_flat, token_table, pe)
    return out.reshape(B, S, D)


# speedup vs baseline: 4.9468x; 1.5946x over previous
import functools

import numpy as np
import jax
import jax.numpy as jnp
from jax import lax
from jax.experimental import pallas as pl
from jax.experimental.pallas import tpu as pltpu
from jax.experimental.pallas import tpu_sc as plsc

_NC, _NS, _LANES = 2, 16, 16
_NW = _NC * _NS
_NBUF = 4


@functools.lru_cache(maxsize=None)
def _sinusoidal_pe(seq_len: int, d_model: int) -> np.ndarray:
    pos = np.arange(seq_len, dtype=np.float32)[:, None]
    two_i = np.arange(0, d_model, 2, dtype=np.float32)
    div = np.power(10000.0, (two_i / np.float32(d_model)).astype(np.float32))
    pe = np.zeros((seq_len, d_model), dtype=np.float32)
    pe[:, 0::2] = np.sin(pos / div)
    pe[:, 1::2] = np.cos(pos / div)
    return pe


@functools.lru_cache(maxsize=None)
def _make_embed(B: int, S: int, V: int, D: int):
    assert S % _NW == 0 and D % _LANES == 0
    s_per_w = S // _NW
    ch = 16
    assert s_per_w % ch == 0 and ch % 8 == 0
    nch = s_per_w // ch
    nu = nch * B
    assert nu % _NBUF == 0

    mesh = plsc.VectorSubcoreMesh(
        core_axis_name="c", subcore_axis_name="s",
        num_cores=_NC, num_subcores=_NS)

    def body(x_hbm, table_hbm, pe_hbm, out_hbm, idx_all,
             r0, r1, r2, r3, pe0, pe1,
             g0, g1, g2, g3, o0, o1, o2, o3, p0, p1):
        rows = [r0, r1, r2, r3]
        gsem = [g0, g1, g2, g3]
        osem = [o0, o1, o2, o3]
        peb, psem = [pe0, pe1], [p0, p1]

        cid = lax.axis_index("c")
        sid = lax.axis_index("s")
        wid = sid * _NC + cid
        s_base = wid * s_per_w

        for b in range(B):
            pltpu.sync_copy(x_hbm.at[pl.ds(b * S + s_base, s_per_w)],
                            idx_all.at[b])

        units = [(c, b) for c in range(nch) for b in range(B)]

        def gather_start(u):
            c, b = units[u]
            p = u % _NBUF
            return pltpu.async_copy(
                table_hbm.at[idx_all.at[b, pl.ds(c * ch, ch)]],
                rows[p], gsem[p])

        def out_start(u):
            c, b = units[u]
            p = u % _NBUF
            return pltpu.async_copy(
                rows[p], out_hbm.at[pl.ds(b * S + s_base + c * ch, ch)],
                osem[p])

        def pe_start(c):
            return pltpu.async_copy(
                pe_hbm.at[pl.ds(s_base + c * ch, ch)], peb[c % 2],
                psem[c % 2])

        pe_descs = {0: pe_start(0)}
        if nch > 1:
            pe_descs[1] = pe_start(1)
        descs = {0: gather_start(0)}
        if nu > 1:
            descs[1] = gather_start(1)
        out_descs = {}

        for u in range(nu):
            c, b = units[u]
            un = u + 2
            if un < nu:
                if un >= _NBUF:
                    out_descs[un - _NBUF].wait()
                descs[un] = gather_start(un)
            descs[u].wait()
            if b == 0:
                pe_descs[c].wait()

            rbuf, pbuf = rows[u % _NBUF], peb[c % 2]

            def add_row(r, _):
                for j in range(D // _LANES):
                    sl = pl.ds(j * _LANES, _LANES)
                    plsc.addupdate(rbuf.at[r, sl], pbuf[r, sl])
                return 0

            lax.fori_loop(0, ch, add_row, 0)

            if b == B - 1 and c + 2 < nch:
                pe_descs[c + 2] = pe_start(c + 2)
            out_descs[u] = out_start(u)

        for u in range(max(0, nu - _NBUF), nu):
            out_descs[u].wait()

    return pl.kernel(
        body,
        out_type=jax.ShapeDtypeStruct((B * S, D), jnp.float32),
        mesh=mesh,
        scratch_types=[
            pltpu.VMEM((B, s_per_w), jnp.int32),
            pltpu.VMEM((ch, D), jnp.float32),
            pltpu.VMEM((ch, D), jnp.float32),
            pltpu.VMEM((ch, D), jnp.float32),
            pltpu.VMEM((ch, D), jnp.float32),
            pltpu.VMEM((ch, D), jnp.float32),
            pltpu.VMEM((ch, D), jnp.float32),
        ] + [pltpu.SemaphoreType.DMA] * 10,
    )


def kernel(x, token_table):
    B, S = x.shape
    V, D = token_table.shape
    pe = jnp.asarray(_sinusoidal_pe(S, D))
    x_flat = x.reshape(B * S).astype(jnp.int32)
    out = _make_embed(B, S, V, D)(x_flat, token_table, pe)
    return out.reshape(B, S, D)
